# baseline (speedup 1.0000x reference)
import jax
import jax.numpy as jnp
from jax.experimental import pallas as pl
from jax.experimental.pallas import tpu as pltpu


_CHUNKS = 5


def _dma_copy_kernel(xu_ref, xi_ref, ou_ref, oi_ref, sems):
    copies = []
    for t, (src, dst) in enumerate(((xu_ref, ou_ref), (xi_ref, oi_ref))):
        n = src.shape[0]
        c = n // _CHUNKS
        for j in range(_CHUNKS):
            lo = j * c
            hi = n - lo if j == _CHUNKS - 1 else c
            cp = pltpu.make_async_copy(
                src.at[pl.ds(lo, hi)],
                dst.at[pl.ds(lo, hi)],
                sems.at[t * _CHUNKS + j],
            )
            cp.start()
            copies.append(cp)
    for cp in copies:
        cp.wait()


def kernel(x_user, x_item, edge_index_user_item, edge_index_item_user):
    del edge_index_user_item, edge_index_item_user
    any_spec = pl.BlockSpec(memory_space=pl.ANY)
    out_u, out_i = pl.pallas_call(
        _dma_copy_kernel,
        in_specs=[any_spec, any_spec],
        out_specs=[any_spec, any_spec],
        out_shape=[
            jax.ShapeDtypeStruct(x_user.shape, x_user.dtype),
            jax.ShapeDtypeStruct(x_item.shape, x_item.dtype),
        ],
        scratch_shapes=[pltpu.SemaphoreType.DMA((2 * _CHUNKS,))],
    )(x_user, x_item)
    return (out_u, out_i)


# speedup vs baseline: 37.5590x; 37.5590x over previous
import jax
import jax.numpy as jnp
from jax.experimental import pallas as pl


_BLOCK_ROWS = 1000


def _copy2_kernel(xu_ref, xi_ref, ou_ref, oi_ref):
    ou_ref[...] = xu_ref[...]
    oi_ref[...] = xi_ref[...]


def kernel(x_user, x_item, edge_index_user_item, edge_index_item_user):
    del edge_index_user_item, edge_index_item_user
    n, d = x_user.shape
    block_rows = _BLOCK_ROWS if n % _BLOCK_ROWS == 0 else n
    grid = (n // block_rows,)
    spec = pl.BlockSpec((block_rows, d), lambda i: (i, 0))
    out_u, out_i = pl.pallas_call(
        _copy2_kernel,
        grid=grid,
        in_specs=[spec, spec],
        out_specs=[spec, spec],
        out_shape=[
            jax.ShapeDtypeStruct(x_user.shape, x_user.dtype),
            jax.ShapeDtypeStruct(x_item.shape, x_item.dtype),
        ],
    )(x_user, x_item)
    return (out_u, out_i)


# speedup vs baseline: 46.6206x; 1.2413x over previous
import jax
import jax.numpy as jnp
from jax.experimental import pallas as pl


_BLOCK_ROWS = 5000


def _copy2_kernel(xu_ref, xi_ref, ou_ref, oi_ref):
    ou_ref[...] = xu_ref[...]
    oi_ref[...] = xi_ref[...]


def kernel(x_user, x_item, edge_index_user_item, edge_index_item_user):
    del edge_index_user_item, edge_index_item_user
    n, d = x_user.shape
    block_rows = _BLOCK_ROWS if n % _BLOCK_ROWS == 0 else n
    grid = (n // block_rows,)
    spec = pl.BlockSpec((block_rows, d), lambda i: (i, 0))
    out_u, out_i = pl.pallas_call(
        _copy2_kernel,
        grid=grid,
        in_specs=[spec, spec],
        out_specs=[spec, spec],
        out_shape=[
            jax.ShapeDtypeStruct(x_user.shape, x_user.dtype),
            jax.ShapeDtypeStruct(x_item.shape, x_item.dtype),
        ],
    )(x_user, x_item)
    return (out_u, out_i)
